# Initial kernel scaffold; baseline (speedup 1.0000x reference)
import jax
import jax.numpy as jnp
from jax.experimental import pallas as pl


def kernel(input_fea, input_loc, W_res, b_res, gamma_res, beta_res, W_off):
    raise NotImplementedError("write your pallas kernel here")


# speedup vs baseline: 15.1716x; 15.1716x over previous
import jax
import jax.numpy as jnp
from jax import lax
from jax.experimental import pallas as pl
from jax.experimental.pallas import tpu as pltpu

B = 16
N = 4096
C = 64
NODES = 64
NSAMP = 64
RAD2 = 0.3 ** 2
MTOT = float(B * N)
NEGBIG = -3.0e38
POSBIG = 3.0e38


def _bi(shape, dim):
    return lax.broadcasted_iota(jnp.int32, shape, dim)


def _eye64():
    return (_bi((64, 64), 0) == _bi((64, 64), 1)).astype(jnp.float32)


def _col(row, eye):
    return lax.dot_general(eye, row, (((1,), (1,)), ((), ())),
                           preferred_element_type=jnp.float32, precision=lax.Precision.HIGHEST)


def _row(colv, eye):
    return lax.dot_general(colv, eye, (((0,), (0,)), ((), ())),
                           preferred_element_type=jnp.float32, precision=lax.Precision.HIGHEST)


def _sqdist(nx, ny, nz, lxr, lyr, lzr):
    bf = jnp.bfloat16
    f32 = jnp.float32
    p0 = nx.astype(bf).astype(f32) * lxr.astype(bf).astype(f32)
    p1 = ny.astype(bf).astype(f32) * lyr.astype(bf).astype(f32)
    p2 = nz.astype(bf).astype(f32) * lzr.astype(bf).astype(f32)
    ab = (p0 + p1) + p2
    n2 = nx * nx + ny * ny + nz * nz
    l2 = lxr * lxr + lyr * lyr + lzr * lzr
    return (-2.0 * ab + n2) + l2


def _k1a_body(loc_ref, fea_ref, woff_ref, fpl_ref, bmat_ref, a_ref,
              s1_ref, s2_ref):
    woff = woff_ref[...]
    s1 = jnp.zeros((C, 1), jnp.float32)
    s2 = jnp.zeros((C, C), jnp.float32)
    for b in range(B):
        fea_b = fea_ref[b]
        a_ref[b] = jnp.dot(woff, fea_b, preferred_element_type=jnp.float32, precision=lax.Precision.HIGHEST)
        s1 = s1 + jnp.sum(fea_b, axis=1, keepdims=True)
        s2 = s2 + lax.dot_general(fea_b, fea_b, (((1,), (1,)), ((), ())),
                                  preferred_element_type=jnp.float32, precision=lax.Precision.HIGHEST)
    s1_ref[...] = s1
    s2_ref[...] = s2

    lx = loc_ref[:, 0, :]
    ly = loc_ref[:, 1, :]
    lz = loc_ref[:, 2, :]
    a0 = a_ref[:, 0, :]
    a1 = a_ref[:, 1, :]
    a2 = a_ref[:, 2, :]
    iota_l = _bi((B, N), 1)
    iota_s = _bi((B, NODES), 1)

    def step(t, carry):
        distance, far, fx, fy, fz, g0, g1, g2 = carry
        onehot = (iota_l == far).astype(jnp.float32)
        cx = jnp.sum(lx * onehot, axis=1, keepdims=True)
        cy = jnp.sum(ly * onehot, axis=1, keepdims=True)
        cz = jnp.sum(lz * onehot, axis=1, keepdims=True)
        b0 = jnp.sum(a0 * onehot, axis=1, keepdims=True)
        b1 = jnp.sum(a1 * onehot, axis=1, keepdims=True)
        b2 = jnp.sum(a2 * onehot, axis=1, keepdims=True)
        sel = (iota_s == t)
        fx = jnp.where(sel, cx, fx)
        fy = jnp.where(sel, cy, fy)
        fz = jnp.where(sel, cz, fz)
        g0 = jnp.where(sel, b0, g0)
        g1 = jnp.where(sel, b1, g1)
        g2 = jnp.where(sel, b2, g2)
        dx = lx - cx
        dy = ly - cy
        dz = lz - cz
        dist = (dx * dx + dy * dy) + dz * dz
        distance = jnp.minimum(distance, dist)
        m = jnp.max(distance, axis=1, keepdims=True)
        far = jnp.min(jnp.where(distance == m, iota_l, N), axis=1,
                      keepdims=True)
        return distance, far, fx, fy, fz, g0, g1, g2

    z = jnp.zeros((B, NODES), jnp.float32)
    carry = (jnp.full((B, N), 1.0e10, jnp.float32),
             jnp.zeros((B, 1), jnp.int32), z, z, z, z, z, z)
    _, _, fx, fy, fz, g0, g1, g2 = lax.fori_loop(0, NODES, step, carry)
    fpl_ref[:, 0, :] = fx
    fpl_ref[:, 1, :] = fy
    fpl_ref[:, 2, :] = fz
    bmat_ref[:, 0, :] = g0
    bmat_ref[:, 1, :] = g1
    bmat_ref[:, 2, :] = g2


def _k1b_body(loc_ref, a_ref, fpl_ref, bmat_ref, nloc_ref, noff_ref):
    eye = _eye64()
    lxr = loc_ref[0, 0:1, :]
    lyr = loc_ref[0, 1:2, :]
    lzr = loc_ref[0, 2:3, :]
    nx = _col(fpl_ref[0, 0:1, :], eye)
    ny = _col(fpl_ref[0, 1:2, :], eye)
    nz = _col(fpl_ref[0, 2:3, :], eye)
    sqr = _sqdist(nx, ny, nz, lxr, lyr, lzr)

    mask = sqr <= RAD2
    iota_l = _bi((NODES, N), 1)
    cnt = jnp.sum(mask.astype(jnp.int32), axis=1, keepdims=True)
    i0 = jnp.min(jnp.where(mask, iota_l, N), axis=1, keepdims=True)

    lo = jnp.full((NODES, 1), -1, jnp.int32)
    hi = jnp.full((NODES, 1), N - 1, jnp.int32)
    for _ in range(12):
        mid = lo + jnp.right_shift(hi - lo, 1)
        c = jnp.sum((mask & (iota_l <= mid)).astype(jnp.int32), axis=1,
                    keepdims=True)
        ge = c >= NSAMP
        hi = jnp.where(ge, mid, hi)
        lo = jnp.where(ge, lo, mid)
    cap = jnp.where(cnt >= NSAMP, hi, N - 1)
    selected = mask & (iota_l <= cap)
    pad = (NSAMP - jnp.minimum(cnt, NSAMP)).astype(jnp.float32)
    w = selected.astype(jnp.float32) + pad * (iota_l == i0).astype(
        jnp.float32)

    bm0 = _col(bmat_ref[0, 0:1, :], eye)
    bm1 = _col(bmat_ref[0, 1:2, :], eye)
    bm2 = _col(bmat_ref[0, 2:3, :], eye)
    a0 = a_ref[0, 0:1, :]
    a1 = a_ref[0, 1:2, :]
    a2 = a_ref[0, 2:3, :]
    inv = 1.0 / float(NSAMP)
    off0 = jnp.sum(w * (jnp.tanh(a0 - bm0) * (lxr - nx)), axis=1,
                   keepdims=True) * inv
    off1 = jnp.sum(w * (jnp.tanh(a1 - bm1) * (lyr - ny)), axis=1,
                   keepdims=True) * inv
    off2 = jnp.sum(w * (jnp.tanh(a2 - bm2) * (lzr - nz)), axis=1,
                   keepdims=True) * inv
    noff_ref[:, 0, :] = _row(off0, eye)
    noff_ref[:, 1, :] = _row(off1, eye)
    noff_ref[:, 2, :] = _row(off2, eye)
    nloc_ref[:, 0, :] = _row(nx + off0, eye)
    nloc_ref[:, 1, :] = _row(ny + off1, eye)
    nloc_ref[:, 2, :] = _row(nz + off2, eye)


def _k2_body(loc_ref, fea_ref, nloc_ref, w_ref, b_ref, gam_ref, bet_ref,
             s1_ref, s2_ref, interp_ref, nfea_ref):
    eye = _eye64()
    W = w_ref[...]
    ws1 = jnp.dot(W, s1_ref[...], preferred_element_type=jnp.float32, precision=lax.Precision.HIGHEST)
    ws2 = jnp.dot(W, s2_ref[...], preferred_element_type=jnp.float32, precision=lax.Precision.HIGHEST)
    diagq = jnp.sum(ws2 * W, axis=1, keepdims=True)
    mean_wx = ws1 / MTOT
    var = diagq / MTOT - mean_wx * mean_wx
    scale = gam_ref[...] / jnp.sqrt(var + 1.0e-5)
    Wh = scale * W
    bh = bet_ref[...] - scale * mean_wx

    fea = fea_ref[0]
    resid = jnp.maximum(
        jnp.dot(Wh, fea, preferred_element_type=jnp.float32, precision=lax.Precision.HIGHEST) + bh, 0.0)

    lxr = loc_ref[0, 0:1, :]
    lyr = loc_ref[0, 1:2, :]
    lzr = loc_ref[0, 2:3, :]
    nx = _col(nloc_ref[0, 0:1, :], eye)
    ny = _col(nloc_ref[0, 1:2, :], eye)
    nz = _col(nloc_ref[0, 2:3, :], eye)
    sqr = _sqdist(nx, ny, nz, lxr, lyr, lzr)

    k = lax.bitcast_convert_type(sqr, jnp.int32)
    k2 = jnp.where(k < 0, k ^ jnp.int32(0x7FFFFFFF), k)
    iota_l = _bi((NODES, N), 1)

    lo = jnp.full((NODES, 1), -980000000, jnp.int32)
    hi = jnp.full((NODES, 1), 1090519040, jnp.int32)
    for _ in range(31):
        mid = lo + jnp.right_shift(hi - lo, 1)
        c = jnp.sum((k2 <= mid).astype(jnp.int32), axis=1, keepdims=True)
        ge = c >= NSAMP
        hi = jnp.where(ge, mid, hi)
        lo = jnp.where(ge, lo, mid)
    T = hi
    ltm = k2 < T
    eqm = k2 == T
    nlt = jnp.sum(ltm.astype(jnp.int32), axis=1, keepdims=True)
    lo2 = jnp.full((NODES, 1), -1, jnp.int32)
    hi2 = jnp.full((NODES, 1), N - 1, jnp.int32)
    for _ in range(12):
        mid = lo2 + jnp.right_shift(hi2 - lo2, 1)
        c = nlt + jnp.sum((eqm & (iota_l <= mid)).astype(jnp.int32),
                          axis=1, keepdims=True)
        ge = c >= NSAMP
        hi2 = jnp.where(ge, mid, hi2)
        lo2 = jnp.where(ge, lo2, mid)
    mask2 = ltm | (eqm & (iota_l <= hi2))

    iota_cl = _bi((NODES, C), 1)
    nfT = jnp.zeros((NODES, C), jnp.float32)
    for c in range(C):
        v = jnp.where(mask2, resid[c:c + 1, :], NEGBIG)
        mx = jnp.max(v, axis=1, keepdims=True)
        nfT = jnp.where(iota_cl == c, mx, nfT)
    nf = lax.dot_general(nfT, eye, (((0,), (0,)), ((), ())),
                         preferred_element_type=jnp.float32, precision=lax.Precision.HIGHEST)
    nfea_ref[0] = nf

    iota_s = _bi((NODES, N), 0)
    d = sqr
    wmT = jnp.zeros((NODES, N), jnp.float32)
    wsum = jnp.zeros((1, N), jnp.float32)
    for _ in range(3):
        m1 = jnp.min(d, axis=0, keepdims=True)
        selidx = jnp.min(jnp.where(d == m1, iota_s, NODES), axis=0,
                         keepdims=True)
        fm = iota_s == selidx
        wk = 1.0 / jnp.maximum(m1, 1.0e-10)
        wmT = wmT + wk * fm.astype(jnp.float32)
        wsum = wsum + wk
        d = jnp.where(fm, POSBIG, d)
    wmT = wmT / wsum
    interp_ref[0] = jnp.dot(nf, wmT, preferred_element_type=jnp.float32, precision=lax.Precision.HIGHEST)


def kernel(input_fea, input_loc, W_res, b_res, gamma_res, beta_res, W_off):
    fea3 = jnp.squeeze(input_fea, 3)

    fpl, bmat, A, s1, s2 = pl.pallas_call(
        _k1a_body,
        out_shape=[
            jax.ShapeDtypeStruct((B, 3, NODES), jnp.float32),
            jax.ShapeDtypeStruct((B, 3, NODES), jnp.float32),
            jax.ShapeDtypeStruct((B, 3, N), jnp.float32),
            jax.ShapeDtypeStruct((C, 1), jnp.float32),
            jax.ShapeDtypeStruct((C, C), jnp.float32),
        ],
    )(input_loc, fea3, W_off)

    node_loc, node_offset = pl.pallas_call(
        _k1b_body,
        grid=(B,),
        in_specs=[
            pl.BlockSpec((1, 3, N), lambda b: (b, 0, 0)),
            pl.BlockSpec((1, 3, N), lambda b: (b, 0, 0)),
            pl.BlockSpec((1, 3, NODES), lambda b: (b, 0, 0)),
            pl.BlockSpec((1, 3, NODES), lambda b: (b, 0, 0)),
        ],
        out_specs=[
            pl.BlockSpec((1, 3, NODES), lambda b: (b, 0, 0)),
            pl.BlockSpec((1, 3, NODES), lambda b: (b, 0, 0)),
        ],
        out_shape=[
            jax.ShapeDtypeStruct((B, 3, NODES), jnp.float32),
            jax.ShapeDtypeStruct((B, 3, NODES), jnp.float32),
        ],
        compiler_params=pltpu.CompilerParams(
            dimension_semantics=("parallel",)),
    )(input_loc, A, fpl, bmat)

    interp, nfea = pl.pallas_call(
        _k2_body,
        grid=(B,),
        in_specs=[
            pl.BlockSpec((1, 3, N), lambda b: (b, 0, 0)),
            pl.BlockSpec((1, C, N), lambda b: (b, 0, 0)),
            pl.BlockSpec((1, 3, NODES), lambda b: (b, 0, 0)),
            pl.BlockSpec((C, C), lambda b: (0, 0)),
            pl.BlockSpec((C, 1), lambda b: (0, 0)),
            pl.BlockSpec((C, 1), lambda b: (0, 0)),
            pl.BlockSpec((C, 1), lambda b: (0, 0)),
            pl.BlockSpec((C, 1), lambda b: (0, 0)),
            pl.BlockSpec((C, C), lambda b: (0, 0)),
        ],
        out_specs=[
            pl.BlockSpec((1, C, N), lambda b: (b, 0, 0)),
            pl.BlockSpec((1, C, NODES), lambda b: (b, 0, 0)),
        ],
        out_shape=[
            jax.ShapeDtypeStruct((B, C, N), jnp.float32),
            jax.ShapeDtypeStruct((B, C, NODES), jnp.float32),
        ],
        compiler_params=pltpu.CompilerParams(
            dimension_semantics=("parallel",)),
    )(input_loc, fea3, node_loc, W_res, b_res.reshape(C, 1),
      gamma_res.reshape(C, 1), beta_res.reshape(C, 1), s1, s2)

    output_fea = jnp.concatenate([fea3, interp], axis=1)[..., None]
    return output_fea, nfea[..., None], node_offset
